# baseline (speedup 1.0000x reference)
import jax
import jax.numpy as jnp
from jax import lax
from jax.experimental import pallas as pl
from jax.experimental.pallas import tpu as pltpu
from jax.experimental.pallas import tpu_sc as plsc

V, D, B, L = 100000, 128, 4096, 50
NC, NS, LANES = 2, 16, 16
NW = NC * NS
BPW = B // NW
C = 4
ROWS = C * L
NCH = BPW // C
NBUF = 4
KD = D // LANES
PAD = 17
GROUPS = (0, 16, 32)


def _body(center_hbm, ctx_hbm, in_hbm, out_hbm, score_hbm,
          cidx_all, ctx_idx_all, vrows, urows, score_v, sems, osems):
    wid = lax.axis_index("s") * NC + lax.axis_index("c")
    iota = lax.iota(jnp.int32, LANES)
    SCP = ROWS + LANES

    pltpu.sync_copy(center_hbm.at[pl.ds(wid * BPW, BPW)], cidx_all)
    pltpu.sync_copy(ctx_hbm.at[pl.ds(wid * BPW * L, BPW * L)], ctx_idx_all)
    pltpu.async_copy(in_hbm.at[cidx_all], vrows, sems.at[0]).wait()

    def issue(ch, buf):
        pltpu.async_copy(out_hbm.at[ctx_idx_all.at[pl.ds(ch * ROWS, ROWS)]],
                         urows.at[pl.ds(buf * ROWS, ROWS)], sems.at[buf])

    def wait(buf):
        pltpu.make_async_copy(out_hbm.at[pl.ds(0, ROWS)],
                              urows.at[pl.ds(buf * ROWS, ROWS)],
                              sems.at[buf]).wait()

    for p in range(NBUF - 1):
        issue(p, p)

    @pl.loop(0, NCH)
    def _outer(ch):
        buf = lax.rem(ch, NBUF)

        @pl.when(ch + NBUF - 1 < NCH)
        def _prefetch():
            issue(ch + NBUF - 1, lax.rem(ch + NBUF - 1, NBUF))

        wait(buf)
        ub = buf * ROWS
        sb = buf * SCP

        @pl.when(ch >= NBUF)
        def _reclaim():
            pltpu.make_async_copy(score_v.at[pl.ds(sb, ROWS)],
                                  score_hbm.at[pl.ds(0, ROWS)],
                                  osems.at[buf]).wait()

        @pl.loop(0, C)
        def _b(b):
            vvecs = [vrows[ch * C + b, pl.ds(k * LANES, LANES)]
                     for k in range(KD)]
            for s in GROUPS:
                score_vec = jnp.zeros((LANES,), jnp.float32)
                for r in range(LANES):
                    row = ub + b * L + s + r
                    prods = [vvecs[k] * urows[row, pl.ds(k * LANES, LANES)]
                             for k in range(KD)]
                    while len(prods) > 1:
                        prods = [prods[i] + prods[i + 1]
                                 for i in range(0, len(prods), 2)]
                    score_vec = jnp.where(iota == r, jnp.sum(prods[0]),
                                          score_vec)
                score_v[pl.ds(sb + b * L + s, LANES)] = score_vec
            tail = jnp.zeros((LANES,), jnp.float32)
            for r in range(2):
                row = ub + b * L + 48 + r
                prods = [vvecs[k] * urows[row, pl.ds(k * LANES, LANES)]
                         for k in range(KD)]
                while len(prods) > 1:
                    prods = [prods[i] + prods[i + 1]
                             for i in range(0, len(prods), 2)]
                tail = jnp.where(iota == r, jnp.sum(prods[0]), tail)
            score_v[pl.ds(sb + b * L + 48, LANES)] = tail

        pltpu.async_copy(score_v.at[pl.ds(sb, ROWS)],
                         score_hbm.at[pl.ds((wid * BPW + ch * C) * L, ROWS)],
                         osems.at[buf])

    for p in range(NBUF):
        pltpu.make_async_copy(score_v.at[pl.ds(p * (ROWS + LANES), ROWS)],
                              score_hbm.at[pl.ds(0, ROWS)],
                              osems.at[p]).wait()


def kernel(center, context, in_em, out_em):
    ctx_flat = context.reshape(B * L).astype(jnp.int32)
    center32 = center.astype(jnp.int32)
    mesh = plsc.VectorSubcoreMesh(core_axis_name="c", subcore_axis_name="s")
    score = pl.kernel(
        _body,
        out_type=jax.ShapeDtypeStruct((B * L,), jnp.float32),
        mesh=mesh,
        compiler_params=pltpu.CompilerParams(needs_layout_passes=False),
        scratch_types=[
            pltpu.VMEM((BPW,), jnp.int32),
            pltpu.VMEM((BPW * L,), jnp.int32),
            pltpu.VMEM((BPW, D), jnp.float32),
            pltpu.VMEM((NBUF * ROWS, D), jnp.float32),
            pltpu.VMEM((NBUF * (ROWS + LANES),), jnp.float32),
            pltpu.SemaphoreType.DMA((NBUF,)),
            pltpu.SemaphoreType.DMA((NBUF,)),
        ],
    )(center32, ctx_flat, in_em, out_em)
    return score.reshape(B, L)
